# baseline (speedup 1.0000x reference)
import functools

import numpy as np
import jax
import jax.numpy as jnp
from jax import lax
from jax.experimental import pallas as pl
from jax.experimental.pallas import tpu as pltpu

_KSIZE = 25
_NLAYERS = 2
_EPS = 1e-5


def _decomp_operator(seq_len, kernel_size, num_layers):
    n_front = kernel_size - 1 - (kernel_size - 1) // 2
    m = np.zeros((seq_len, seq_len), dtype=np.float64)
    for row in range(seq_len):
        for t in range(row, row + kernel_size):
            j = min(max(t - n_front, 0), seq_len - 1)
            m[row, j] += 1.0 / kernel_size
    season = np.linalg.matrix_power(np.eye(seq_len) - m, num_layers)
    return np.eye(seq_len) - season


_ODEPTH = 3


def _fused_kernel(x_ref, tr_ref, r_ref, w_ref, col_ref,
                  b_ref, out_ref, trout_hbm, tobuf, osems, *, eps, tb, nsteps):
    r_mat = r_ref[...]
    w = w_ref[...]
    col = col_ref[...]
    bias = b_ref[...]
    k = pl.program_id(0)
    slot = jax.lax.rem(k, _ODEPTH)

    def _ring_copy(d):
        return pltpu.make_async_copy(
            tobuf.at[d], trout_hbm.at[pl.ds(k * tb, tb)], osems.at[d])

    @pl.when(k >= _ODEPTH)
    def _():
        _ring_copy(slot).wait()

    for i in range(tb):
        a = x_ref[i]
        z = jnp.dot(r_mat, a.astype(jnp.bfloat16),
                    preferred_element_type=jnp.float32)
        res = a - z
        tobuf[slot, i] = tr_ref[i] + z
        mu = jnp.mean(res, axis=-1, keepdims=True)
        ex2 = jnp.mean(res * res, axis=-1, keepdims=True)
        rsig = lax.rsqrt(ex2 - mu * mu + eps)
        y = jnp.dot(res.astype(jnp.bfloat16), w,
                    preferred_element_type=jnp.float32)
        out_ref[i] = rsig * (y - mu * col) + bias

    _ring_copy(slot).start()

    @pl.when(k == nsteps - 1)
    def _():
        for d in range(min(_ODEPTH, nsteps)):
            _ring_copy(d).wait()


def kernel(x, cross, trend, gamma, beta, w_proj, b_proj):
    del cross
    batch, seq_len, chan = x.shape
    c_out = w_proj.shape[1]

    trendm = _decomp_operator(seq_len, _KSIZE, _NLAYERS)
    r_mat = jnp.asarray(trendm.astype(np.float32)).astype(jnp.bfloat16)

    w_s = gamma.reshape(chan, 1) * w_proj
    col = jnp.sum(w_s, axis=0, keepdims=True)
    b_f = b_proj.reshape(1, c_out) + beta.reshape(1, chan) @ w_proj
    w_s = w_s.astype(jnp.bfloat16)

    tb = 16 if batch % 16 == 0 else (8 if batch % 8 == 0 else
                                     (2 if batch % 2 == 0 else 1))
    grid = (batch // tb,)

    row_blk = pl.BlockSpec((tb, seq_len, chan), lambda i: (i, 0, 0))
    out, trend_out = pl.pallas_call(
        functools.partial(_fused_kernel, eps=_EPS, tb=tb,
                          nsteps=batch // tb),
        out_shape=(jax.ShapeDtypeStruct((batch, seq_len, c_out), x.dtype),
                   jax.ShapeDtypeStruct((batch, seq_len, chan), trend.dtype)),
        grid=grid,
        in_specs=[
            row_blk,
            row_blk,
            pl.BlockSpec((seq_len, seq_len), lambda i: (0, 0)),
            pl.BlockSpec((chan, c_out), lambda i: (0, 0)),
            pl.BlockSpec((1, c_out), lambda i: (0, 0)),
            pl.BlockSpec((1, c_out), lambda i: (0, 0)),
        ],
        out_specs=(pl.BlockSpec((tb, seq_len, c_out), lambda i: (i, 0, 0)),
                   pl.BlockSpec(memory_space=pl.ANY)),
        scratch_shapes=[
            pltpu.VMEM((_ODEPTH, tb, seq_len, chan), trend.dtype),
            pltpu.SemaphoreType.DMA((_ODEPTH,)),
        ],
        compiler_params=pltpu.CompilerParams(
            dimension_semantics=("arbitrary",)),
    )(x, trend, r_mat, w_s, col, b_f)
    return out, trend_out


# speedup vs baseline: 1.0097x; 1.0097x over previous
import functools

import numpy as np
import jax
import jax.numpy as jnp
from jax import lax
from jax.experimental import pallas as pl
from jax.experimental.pallas import tpu as pltpu

_KSIZE = 25
_NLAYERS = 2
_EPS = 1e-5


def _decomp_operator(seq_len, kernel_size, num_layers):
    n_front = kernel_size - 1 - (kernel_size - 1) // 2
    m = np.zeros((seq_len, seq_len), dtype=np.float64)
    for row in range(seq_len):
        for t in range(row, row + kernel_size):
            j = min(max(t - n_front, 0), seq_len - 1)
            m[row, j] += 1.0 / kernel_size
    season = np.linalg.matrix_power(np.eye(seq_len) - m, num_layers)
    return np.eye(seq_len) - season


def _fused_kernel(x_ref, tr_ref, r_ref, w_ref, col_ref,
                  b_ref, out_ref, trout_ref, *, eps, tb):
    r_mat = r_ref[...]
    w = w_ref[...]
    col = col_ref[...]
    bias = b_ref[...]
    for i in range(tb):
        a = x_ref[i]
        z = jnp.dot(r_mat, a.astype(jnp.bfloat16),
                    preferred_element_type=jnp.float32)
        res = a - z
        trout_ref[i] = tr_ref[i] + z
        mu = jnp.mean(res, axis=-1, keepdims=True)
        ex2 = jnp.mean(res * res, axis=-1, keepdims=True)
        rsig = lax.rsqrt(ex2 - mu * mu + eps)
        y = jnp.dot(res.astype(jnp.bfloat16), w,
                    preferred_element_type=jnp.float32)
        out_ref[i] = rsig * (y - mu * col) + bias


def kernel(x, cross, trend, gamma, beta, w_proj, b_proj):
    del cross
    batch, seq_len, chan = x.shape
    c_out = w_proj.shape[1]

    trendm = _decomp_operator(seq_len, _KSIZE, _NLAYERS)
    r_mat = jnp.asarray(trendm.astype(np.float32)).astype(jnp.bfloat16)

    w_s = gamma.reshape(chan, 1) * w_proj
    col = jnp.sum(w_s, axis=0, keepdims=True)
    b_f = b_proj.reshape(1, c_out) + beta.reshape(1, chan) @ w_proj
    w_s = w_s.astype(jnp.bfloat16)

    tb = 16 if batch % 16 == 0 else (8 if batch % 8 == 0 else
                                     (2 if batch % 2 == 0 else 1))
    grid = (batch // tb,)

    row_blk = pl.BlockSpec((tb, seq_len, chan), lambda i: (i, 0, 0))
    out, trend_out = pl.pallas_call(
        functools.partial(_fused_kernel, eps=_EPS, tb=tb),
        out_shape=(jax.ShapeDtypeStruct((batch, seq_len, c_out), x.dtype),
                   jax.ShapeDtypeStruct((batch, seq_len, chan), trend.dtype)),
        grid=grid,
        in_specs=[
            row_blk,
            row_blk,
            pl.BlockSpec((seq_len, seq_len), lambda i: (0, 0)),
            pl.BlockSpec((chan, c_out), lambda i: (0, 0)),
            pl.BlockSpec((1, c_out), lambda i: (0, 0)),
            pl.BlockSpec((1, c_out), lambda i: (0, 0)),
        ],
        out_specs=(pl.BlockSpec((tb, seq_len, c_out), lambda i: (i, 0, 0)),
                   row_blk),
        compiler_params=pltpu.CompilerParams(
            dimension_semantics=("parallel",)),
    )(x, trend, r_mat, w_s, col, b_f)
    return out, trend_out
